# Initial kernel scaffold; baseline (speedup 1.0000x reference)
import jax
import jax.numpy as jnp
from jax.experimental import pallas as pl


def kernel(xyz, img, ct_size):
    raise NotImplementedError("write your pallas kernel here")


# speedup vs baseline: 1.0433x; 1.0433x over previous
import functools

import jax
import jax.numpy as jnp
from jax import lax
from jax.experimental import pallas as pl
from jax.experimental.pallas import tpu as pltpu
from jax.experimental.pallas import tpu_sc as plsc

_XL, _YL, _ZL = 511, 511, 255
_WATER = 0.08

_N = 1048576
_NC = 2
_NS = 16
_NW = _NC * _NS
_P = _N // _NW
_C = 8192
_NCH = _P // _C
_G = 128
_R = _C // _G
_FIRE = 8


def _sc_body(xs, ys, zs, par, img, out, xv, yv, zv, pv, idxb, vb, gb, sem):
    wid = lax.axis_index("s") * _NC + lax.axis_index("c")
    base = wid * _P
    pltpu.sync_copy(par, pv)

    def chunk(k, _):
        off = base + k * _C
        pltpu.sync_copy(xs.at[pl.ds(off, _C)], xv)
        pltpu.sync_copy(ys.at[pl.ds(off, _C)], yv)
        pltpu.sync_copy(zs.at[pl.ds(off, _C)], zv)

        def ixloop(j, _):
            s = pl.ds(j * 16, 16)
            px = (xv[s] + pv[0]) / pv[3] * jnp.float32(_XL)
            py = (yv[s] + pv[1]) / pv[4] * jnp.float32(_YL)
            pz = (zv[s] + pv[2]) / pv[5] * jnp.float32(_ZL)
            ixi = px.astype(jnp.int32)
            iyi = py.astype(jnp.int32)
            izi = pz.astype(jnp.int32)
            m = ((ixi < 0) | (iyi < 0) | (izi < 0)
                 | (ixi > _XL) | (iyi > _YL) | (izi > _ZL))
            lin = (ixi * 512 + iyi) * 256 + izi
            idxb[s] = jnp.where(m, 0, lin)
            vb[s] = jnp.where(m, jnp.float32(0.0), jnp.float32(1.0))
            return 0

        lax.fori_loop(0, _C // 16, ixloop, 0)

        def gloop(g, _):
            b = g * _FIRE * _G
            cps = [
                pltpu.async_copy(
                    img.at[idxb.at[pl.ds(b + t * _G, _G)]],
                    gb.at[pl.ds(b + t * _G, _G)],
                    sem,
                )
                for t in range(_FIRE)
            ]
            for cp in cps:
                cp.wait()
            return 0

        lax.fori_loop(0, _R // _FIRE, gloop, 0)

        def trloop(j, _):
            s = pl.ds(j * 16, 16)
            t = jnp.maximum(gb[s], jnp.float32(-1000.0)) / jnp.float32(1000.0)
            gb[s] = (t + jnp.float32(1.0)) * jnp.float32(_WATER) * vb[s]
            return 0

        lax.fori_loop(0, _C // 16, trloop, 0)
        pltpu.sync_copy(gb, out.at[pl.ds(off, _C)])
        return 0

    lax.fori_loop(0, _NCH, chunk, 0)


_sc_gather = functools.partial(
    pl.kernel,
    out_type=jax.ShapeDtypeStruct((_N,), jnp.float32),
    mesh=plsc.VectorSubcoreMesh(core_axis_name="c", subcore_axis_name="s"),
    scratch_types=[
        pltpu.VMEM((_C,), jnp.float32),
        pltpu.VMEM((_C,), jnp.float32),
        pltpu.VMEM((_C,), jnp.float32),
        pltpu.VMEM((6, 16), jnp.float32),
        pltpu.VMEM((_C,), jnp.int32),
        pltpu.VMEM((_C,), jnp.float32),
        pltpu.VMEM((_C,), jnp.float32),
        pltpu.SemaphoreType.DMA,
    ],
)(_sc_body)


def kernel(xyz, img, ct_size):
    pts = xyz[0]
    xs = pts[:, 0]
    ys = pts[:, 1]
    zs = pts[:, 2]
    img_flat = img.reshape(-1)
    half = ct_size / 2.0
    par = jnp.broadcast_to(
        jnp.concatenate([half, ct_size]).astype(jnp.float32).reshape(6, 1),
        (6, 16),
    )
    sigma = _sc_gather(xs, ys, zs, par, img_flat)
    rgb = jnp.ones((1, _N, 3), jnp.float32)
    return jnp.concatenate([rgb, sigma.reshape(1, _N, 1)], axis=-1)
